# baseline (speedup 1.0000x reference)
import functools

import jax
import jax.numpy as jnp
from jax import lax
from jax.experimental import pallas as pl
from jax.experimental.pallas import tpu as pltpu
from jax.experimental.pallas import tpu_sc as plsc

N = 10000
E = 160000
F = 256
HF = 128
L = 4

NCORES = 2
NTILES = 16
CHUNK = 64
CPT = 160
E_PAD = NTILES * CPT * CHUNK
NPASS = 4
CPP = CPT // NPASS
ZROWS = 632
ACC_ROWS = NTILES * ZROWS
TRASH = N


NBUF = 4


def _agg_body(h_ref, src_ref, dst_ref, z_ref, out_ref, src_v, dst_v, rows_v,
              acc, *gsems):
    c = lax.axis_index("c")
    s = lax.axis_index("s")
    pltpu.sync_copy(z_ref, acc.at[pl.ds(s * ZROWS, ZROWS)])
    plsc.subcore_barrier()

    def pass_body(p, carry):
        base = pl.multiple_of(s * CPT + p * CPP, 8)
        pltpu.sync_copy(src_ref.at[c, pl.ds(base, CPP)], src_v)
        pltpu.sync_copy(dst_ref.at[pl.ds(base, CPP)], dst_v)
        for b in range(NBUF):
            pltpu.async_copy(h_ref.at[src_v.at[b]], rows_v.at[b], gsems[b])

        def group(g, carry2):
            for b in range(NBUF):
                j = g * NBUF + b
                pltpu.make_async_copy(h_ref.at[src_v.at[j]], rows_v.at[b],
                                      gsems[b]).wait()
                pltpu.sync_copy(rows_v.at[b], acc.at[dst_v.at[j]], add=True)

                @pl.when(j + NBUF < CPP)
                def _():
                    pltpu.async_copy(h_ref.at[src_v.at[j + NBUF]],
                                     rows_v.at[b], gsems[b])
            return carry2

        lax.fori_loop(0, CPP // NBUF, group, 0)
        return carry

    lax.fori_loop(0, NPASS, pass_body, 0)
    plsc.subcore_barrier()
    pltpu.sync_copy(acc.at[pl.ds(s * ZROWS, ZROWS)],
                    out_ref.at[c, pl.ds(s * ZROWS, ZROWS)])


_agg_call = functools.partial(
    pl.kernel,
    out_type=jax.ShapeDtypeStruct((NCORES, ACC_ROWS, HF), jnp.float32),
    mesh=plsc.VectorSubcoreMesh(core_axis_name="c", subcore_axis_name="s"),
    scratch_types=[
        pltpu.VMEM((CPP, CHUNK), jnp.int32),
        pltpu.VMEM((CPP, CHUNK), jnp.int32),
        pltpu.VMEM((NBUF, CHUNK, HF), jnp.float32),
        pltpu.VMEM_SHARED((ACC_ROWS, HF), jnp.float32),
    ] + [pltpu.SemaphoreType.DMA] * NBUF,
)(_agg_body)


BN_ROWS = 1000


def _layer_body(eps_ref, h_ref, a_ref, w1_ref, c1_ref, w2_ref, c2_ref,
                sp_ref, tp_ref, o_ref, *, last):
    h = jnp.concatenate([h_ref[0], h_ref[1]], axis=1)
    agg = jnp.concatenate([a_ref[0], a_ref[1]], axis=1)
    z = (1.0 + eps_ref[0, 0]) * h + agg
    z = jnp.dot(z, w1_ref[...], preferred_element_type=jnp.float32) + c1_ref[...]
    z = jnp.maximum(z, 0.0)
    u = jnp.dot(z, w2_ref[...], preferred_element_type=jnp.float32) + c2_ref[...]
    if not last:
        u = jnp.maximum(u, 0.0)
        u = u * sp_ref[...] + tp_ref[...]
        u = jnp.maximum(u, 0.0)
    h2 = u + h
    o_ref[0] = h2[:, :HF]
    o_ref[1] = h2[:, HF:]


def _make_layer_call(last):
    body = functools.partial(_layer_body, last=last)
    block = pl.BlockSpec((NCORES, BN_ROWS, HF), lambda i: (0, i, 0))
    full2 = pl.BlockSpec((F, F), lambda i: (0, 0))
    vec = pl.BlockSpec((1, F), lambda i: (0, 0))
    return pl.pallas_call(
        body,
        grid=(N // BN_ROWS,),
        in_specs=[
            pl.BlockSpec(memory_space=pltpu.SMEM),
            block, block, full2, vec, full2, vec, vec, vec,
        ],
        out_specs=block,
        out_shape=jax.ShapeDtypeStruct((NCORES, N, HF), jnp.float32),
    )


_layer_mid = _make_layer_call(last=False)
_layer_last = _make_layer_call(last=True)


def _ln(x, g, b):
    mu = jnp.mean(x, axis=-1, keepdims=True)
    xc = x - mu
    var = jnp.mean(xc * xc, axis=-1, keepdims=True)
    return xc * lax.rsqrt(var + 1e-5) * g + b


def _head_body(h_ref, lng_ref, lnb_ref, w1_ref, b1_ref, g1_ref, t1_ref,
               w2_ref, b2_ref, g2_ref, t2_ref, o_ref):
    h = jnp.concatenate([h_ref[0], h_ref[1]], axis=1)
    h = _ln(h, lng_ref[...], lnb_ref[...])
    a = jnp.dot(h, w1_ref[...], preferred_element_type=jnp.float32) + b1_ref[...]
    a = jnp.maximum(_ln(a, g1_ref[...], t1_ref[...]), 0.0)
    o = jnp.dot(a, w2_ref[...], preferred_element_type=jnp.float32) + b2_ref[...]
    o_ref[...] = jnp.maximum(_ln(o, g2_ref[...], t2_ref[...]), 0.0)


_head_call = pl.pallas_call(
    _head_body,
    grid=(N // BN_ROWS,),
    in_specs=[
        pl.BlockSpec((NCORES, BN_ROWS, HF), lambda i: (0, i, 0)),
        pl.BlockSpec((1, F), lambda i: (0, 0)),
        pl.BlockSpec((1, F), lambda i: (0, 0)),
        pl.BlockSpec((F, 2 * F), lambda i: (0, 0)),
        pl.BlockSpec((1, 2 * F), lambda i: (0, 0)),
        pl.BlockSpec((1, 2 * F), lambda i: (0, 0)),
        pl.BlockSpec((1, 2 * F), lambda i: (0, 0)),
        pl.BlockSpec((2 * F, F), lambda i: (0, 0)),
        pl.BlockSpec((1, F), lambda i: (0, 0)),
        pl.BlockSpec((1, F), lambda i: (0, 0)),
        pl.BlockSpec((1, F), lambda i: (0, 0)),
    ],
    out_specs=pl.BlockSpec((BN_ROWS, F), lambda i: (i, 0)),
    out_shape=jax.ShapeDtypeStruct((N, F), jnp.float32),
)


def _bn_affine(p):
    s = p["gamma"] * lax.rsqrt(p["var"] + 1e-5)
    t = p["beta"] - p["mean"] * s
    return s, t


def kernel(x, edge_index, params):
    pad = E_PAD - E
    src = jnp.concatenate([edge_index[0], jnp.zeros((pad,), jnp.int32)])
    dst = jnp.concatenate([edge_index[1], jnp.full((pad,), TRASH, jnp.int32)])
    src2 = src.reshape(NTILES * CPT, CHUNK)
    src_both = jnp.stack([src2, src2 + N])
    dst2 = dst.reshape(NTILES * CPT, CHUNK)
    zeros = jnp.zeros((ZROWS, HF), jnp.float32)

    H = jnp.stack([x[:, :HF], x[:, HF:]])

    for i in range(L):
        c = params["convs"][i]
        s1, t1 = _bn_affine(c["bn1"])
        w1 = c["W1"] * s1[None, :]
        c1 = (c["b1"] * s1 + t1)[None, :]
        s2, t2 = _bn_affine(c["bn2"])
        w2 = c["W2"] * s2[None, :]
        c2 = c["b2"] * s2 + t2
        sp, tp = _bn_affine(params["post_bn"][i])
        last = i == L - 1
        if last:
            w2 = w2 * sp[None, :]
            c2 = c2 * sp + tp
        epsw = (params["eps"][i]).reshape(1, 1)

        agg = _agg_call(H.reshape(NCORES * N, HF), src_both, dst2, zeros)
        call = _layer_last if last else _layer_mid
        H = call(epsw, H, agg, w1, c1, w2, c2[None, :], sp[None, :], tp[None, :])

    return _head_call(
        H,
        params["ln_g"][None, :], params["ln_b"][None, :],
        params["Wf1"], params["bf1"][None, :],
        params["lnf1_g"][None, :], params["lnf1_b"][None, :],
        params["Wf2"], params["bf2"][None, :],
        params["lnf2_g"][None, :], params["lnf2_b"][None, :],
    )


# speedup vs baseline: 1.0494x; 1.0494x over previous
import functools

import jax
import jax.numpy as jnp
from jax import lax
from jax.experimental import pallas as pl
from jax.experimental.pallas import tpu as pltpu
from jax.experimental.pallas import tpu_sc as plsc

N = 10000
E = 160000
F = 256
HF = 128
L = 4

NCORES = 2
NTILES = 16
CHUNK = 128
CPT = 80
E_PAD = NTILES * CPT * CHUNK
NPASS = 2
CPP = CPT // NPASS
ZROWS = 632
ACC_ROWS = NTILES * ZROWS
TRASH = N


NBUF = 2


def _agg_body(h_ref, src_ref, dst_ref, z_ref, out_ref, src_v, dst_v, rows_v,
              acc, *gsems):
    c = lax.axis_index("c")
    s = lax.axis_index("s")
    pltpu.sync_copy(z_ref, acc.at[pl.ds(s * ZROWS, ZROWS)])
    plsc.subcore_barrier()

    def pass_body(p, carry):
        base = pl.multiple_of(s * CPT + p * CPP, 8)
        pltpu.sync_copy(src_ref.at[c, pl.ds(base, CPP)], src_v)
        pltpu.sync_copy(dst_ref.at[pl.ds(base, CPP)], dst_v)
        for b in range(NBUF):
            pltpu.async_copy(h_ref.at[src_v.at[b]], rows_v.at[b], gsems[b])

        def group(g, carry2):
            for b in range(NBUF):
                j = g * NBUF + b
                pltpu.make_async_copy(h_ref.at[src_v.at[j]], rows_v.at[b],
                                      gsems[b]).wait()
                pltpu.sync_copy(rows_v.at[b], acc.at[dst_v.at[j]], add=True)

                @pl.when(j + NBUF < CPP)
                def _():
                    pltpu.async_copy(h_ref.at[src_v.at[j + NBUF]],
                                     rows_v.at[b], gsems[b])
            return carry2

        lax.fori_loop(0, CPP // NBUF, group, 0)
        return carry

    lax.fori_loop(0, NPASS, pass_body, 0)
    plsc.subcore_barrier()
    pltpu.sync_copy(acc.at[pl.ds(s * ZROWS, ZROWS)],
                    out_ref.at[c, pl.ds(s * ZROWS, ZROWS)])


_agg_call = functools.partial(
    pl.kernel,
    out_type=jax.ShapeDtypeStruct((NCORES, ACC_ROWS, HF), jnp.float32),
    mesh=plsc.VectorSubcoreMesh(core_axis_name="c", subcore_axis_name="s"),
    scratch_types=[
        pltpu.VMEM((CPP, CHUNK), jnp.int32),
        pltpu.VMEM((CPP, CHUNK), jnp.int32),
        pltpu.VMEM((NBUF, CHUNK, HF), jnp.float32),
        pltpu.VMEM_SHARED((ACC_ROWS, HF), jnp.float32),
    ] + [pltpu.SemaphoreType.DMA] * NBUF,
)(_agg_body)


BN_ROWS = 1000


def _layer_body(eps_ref, h_ref, a_ref, w1_ref, c1_ref, w2_ref, c2_ref,
                sp_ref, tp_ref, o_ref, *, last):
    h = jnp.concatenate([h_ref[0], h_ref[1]], axis=1)
    agg = jnp.concatenate([a_ref[0], a_ref[1]], axis=1)
    z = (1.0 + eps_ref[0, 0]) * h + agg
    z = jnp.dot(z, w1_ref[...], preferred_element_type=jnp.float32) + c1_ref[...]
    z = jnp.maximum(z, 0.0)
    u = jnp.dot(z, w2_ref[...], preferred_element_type=jnp.float32) + c2_ref[...]
    if not last:
        u = jnp.maximum(u, 0.0)
        u = u * sp_ref[...] + tp_ref[...]
        u = jnp.maximum(u, 0.0)
    h2 = u + h
    o_ref[0] = h2[:, :HF]
    o_ref[1] = h2[:, HF:]


def _make_layer_call(last):
    body = functools.partial(_layer_body, last=last)
    block = pl.BlockSpec((NCORES, BN_ROWS, HF), lambda i: (0, i, 0))
    full2 = pl.BlockSpec((F, F), lambda i: (0, 0))
    vec = pl.BlockSpec((1, F), lambda i: (0, 0))
    return pl.pallas_call(
        body,
        grid=(N // BN_ROWS,),
        in_specs=[
            pl.BlockSpec(memory_space=pltpu.SMEM),
            block, block, full2, vec, full2, vec, vec, vec,
        ],
        out_specs=block,
        out_shape=jax.ShapeDtypeStruct((NCORES, N, HF), jnp.float32),
    )


_layer_mid = _make_layer_call(last=False)
_layer_last = _make_layer_call(last=True)


def _ln(x, g, b):
    mu = jnp.mean(x, axis=-1, keepdims=True)
    xc = x - mu
    var = jnp.mean(xc * xc, axis=-1, keepdims=True)
    return xc * lax.rsqrt(var + 1e-5) * g + b


def _head_body(h_ref, lng_ref, lnb_ref, w1_ref, b1_ref, g1_ref, t1_ref,
               w2_ref, b2_ref, g2_ref, t2_ref, o_ref):
    h = jnp.concatenate([h_ref[0], h_ref[1]], axis=1)
    h = _ln(h, lng_ref[...], lnb_ref[...])
    a = jnp.dot(h, w1_ref[...], preferred_element_type=jnp.float32) + b1_ref[...]
    a = jnp.maximum(_ln(a, g1_ref[...], t1_ref[...]), 0.0)
    o = jnp.dot(a, w2_ref[...], preferred_element_type=jnp.float32) + b2_ref[...]
    o_ref[...] = jnp.maximum(_ln(o, g2_ref[...], t2_ref[...]), 0.0)


_head_call = pl.pallas_call(
    _head_body,
    grid=(N // BN_ROWS,),
    in_specs=[
        pl.BlockSpec((NCORES, BN_ROWS, HF), lambda i: (0, i, 0)),
        pl.BlockSpec((1, F), lambda i: (0, 0)),
        pl.BlockSpec((1, F), lambda i: (0, 0)),
        pl.BlockSpec((F, 2 * F), lambda i: (0, 0)),
        pl.BlockSpec((1, 2 * F), lambda i: (0, 0)),
        pl.BlockSpec((1, 2 * F), lambda i: (0, 0)),
        pl.BlockSpec((1, 2 * F), lambda i: (0, 0)),
        pl.BlockSpec((2 * F, F), lambda i: (0, 0)),
        pl.BlockSpec((1, F), lambda i: (0, 0)),
        pl.BlockSpec((1, F), lambda i: (0, 0)),
        pl.BlockSpec((1, F), lambda i: (0, 0)),
    ],
    out_specs=pl.BlockSpec((BN_ROWS, F), lambda i: (i, 0)),
    out_shape=jax.ShapeDtypeStruct((N, F), jnp.float32),
)


def _bn_affine(p):
    s = p["gamma"] * lax.rsqrt(p["var"] + 1e-5)
    t = p["beta"] - p["mean"] * s
    return s, t


def kernel(x, edge_index, params):
    pad = E_PAD - E
    src = jnp.concatenate([edge_index[0], jnp.zeros((pad,), jnp.int32)])
    pad_dst = TRASH + (jnp.arange(pad, dtype=jnp.int32) % (ACC_ROWS - N))
    dst = jnp.concatenate([edge_index[1], pad_dst])
    src2 = src.reshape(NTILES * CPT, CHUNK)
    src_both = jnp.stack([src2, src2 + N])
    dst2 = dst.reshape(NTILES * CPT, CHUNK)
    zeros = jnp.zeros((ZROWS, HF), jnp.float32)

    H = jnp.stack([x[:, :HF], x[:, HF:]])

    for i in range(L):
        c = params["convs"][i]
        s1, t1 = _bn_affine(c["bn1"])
        w1 = c["W1"] * s1[None, :]
        c1 = (c["b1"] * s1 + t1)[None, :]
        s2, t2 = _bn_affine(c["bn2"])
        w2 = c["W2"] * s2[None, :]
        c2 = c["b2"] * s2 + t2
        sp, tp = _bn_affine(params["post_bn"][i])
        last = i == L - 1
        if last:
            w2 = w2 * sp[None, :]
            c2 = c2 * sp + tp
        epsw = (params["eps"][i]).reshape(1, 1)

        agg = _agg_call(H.reshape(NCORES * N, HF), src_both, dst2, zeros)
        call = _layer_last if last else _layer_mid
        H = call(epsw, H, agg, w1, c1, w2, c2[None, :], sp[None, :], tp[None, :])

    return _head_call(
        H,
        params["ln_g"][None, :], params["ln_b"][None, :],
        params["Wf1"], params["bf1"][None, :],
        params["lnf1_g"][None, :], params["lnf1_b"][None, :],
        params["Wf2"], params["bf2"][None, :],
        params["lnf2_g"][None, :], params["lnf2_b"][None, :],
    )
